# baseline (speedup 1.0000x reference)
import functools

import jax
import jax.numpy as jnp
from jax.experimental import pallas as pl
from jax.experimental.pallas import tpu as pltpu

_SPLIT = 1
_T = (((1,), (1,)), ((), ()))


def _body(sub, *refs):
    l_refs = refs[:_SPLIT]
    f_ref, w1_ref, w2_ref, b1_ref, b2_ref, out_ref, fb_ref = refs[_SPLIT:]
    i = pl.program_id(0)

    @pl.when(i == 0)
    def _():
        fb_ref[...] = f_ref[...].astype(jnp.bfloat16)

    fb = fb_ref[...]
    w1 = w1_ref[...]
    w2 = w2_ref[...]
    b = b1_ref[...] + b2_ref[...]
    for j, l_ref in enumerate(l_refs):
        x = jnp.dot(
            l_ref[...].astype(jnp.bfloat16),
            fb,
            preferred_element_type=jnp.float32,
        )
        f = f_ref[pl.ds(i * _SPLIT * sub + j * sub, sub), :]
        out_ref[pl.ds(j * sub, sub), :] = (
            jax.lax.dot_general(f + x, w1, _T, preferred_element_type=jnp.float32)
            + jax.lax.dot_general(x * f, w2, _T, preferred_element_type=jnp.float32)
            + b
        )


def kernel(lap_matrix, eye_matrix, features, W1, b1, W2, b2):
    n, d = features.shape
    bm = min(256, n)
    sub = bm // _SPLIT
    nm = n // bm

    l_specs = [
        pl.BlockSpec((sub, n), functools.partial(lambda j, i: (_SPLIT * i + j, 0), j))
        for j in range(_SPLIT)
    ]
    in_specs = l_specs + [
        pl.BlockSpec((n, d), lambda i: (0, 0)),
        pl.BlockSpec((d, d), lambda i: (0, 0)),
        pl.BlockSpec((d, d), lambda i: (0, 0)),
        pl.BlockSpec((1, d), lambda i: (0, 0)),
        pl.BlockSpec((1, d), lambda i: (0, 0)),
    ]

    return pl.pallas_call(
        functools.partial(_body, sub),
        grid=(nm,),
        in_specs=in_specs,
        out_specs=pl.BlockSpec((bm, d), lambda i: (i, 0)),
        out_shape=jax.ShapeDtypeStruct((n, d), jnp.float32),
        scratch_shapes=[pltpu.VMEM((n, d), jnp.bfloat16)],
        compiler_params=pltpu.CompilerParams(
            dimension_semantics=("arbitrary",),
            vmem_limit_bytes=63 * 1024 * 1024,
        ),
    )(*([lap_matrix] * _SPLIT), features, W1, W2,
      b1.reshape(1, d), b2.reshape(1, d))


# speedup vs baseline: 1.0041x; 1.0041x over previous
import functools

import jax
import jax.numpy as jnp
from jax.experimental import pallas as pl
from jax.experimental.pallas import tpu as pltpu

_SPLIT = 1
_T = (((1,), (1,)), ((), ()))


def _body(sub, *refs):
    l_refs = refs[:_SPLIT]
    f_ref, w1_ref, w2_ref, b1_ref, b2_ref, out_ref = refs[_SPLIT:]
    i = pl.program_id(0)
    fb = f_ref[...].astype(jnp.bfloat16)
    w1 = w1_ref[...]
    w2 = w2_ref[...]
    b = b1_ref[...] + b2_ref[...]
    for j, l_ref in enumerate(l_refs):
        x = jnp.dot(
            l_ref[...].astype(jnp.bfloat16),
            fb,
            preferred_element_type=jnp.float32,
        )
        f = f_ref[pl.ds(i * _SPLIT * sub + j * sub, sub), :]
        out_ref[pl.ds(j * sub, sub), :] = (
            jax.lax.dot_general(f + x, w1, _T, preferred_element_type=jnp.float32)
            + jax.lax.dot_general(x * f, w2, _T, preferred_element_type=jnp.float32)
            + b
        )


def kernel(lap_matrix, eye_matrix, features, W1, b1, W2, b2):
    n, d = features.shape
    bm = min(256, n)
    sub = bm // _SPLIT
    nm = n // bm

    l_specs = [
        pl.BlockSpec((sub, n), functools.partial(lambda j, i: (_SPLIT * i + j, 0), j))
        for j in range(_SPLIT)
    ]
    in_specs = l_specs + [
        pl.BlockSpec((n, d), lambda i: (0, 0)),
        pl.BlockSpec((d, d), lambda i: (0, 0)),
        pl.BlockSpec((d, d), lambda i: (0, 0)),
        pl.BlockSpec((1, d), lambda i: (0, 0)),
        pl.BlockSpec((1, d), lambda i: (0, 0)),
    ]

    return pl.pallas_call(
        functools.partial(_body, sub),
        grid=(nm,),
        in_specs=in_specs,
        out_specs=pl.BlockSpec((bm, d), lambda i: (i, 0)),
        out_shape=jax.ShapeDtypeStruct((n, d), jnp.float32),
        compiler_params=pltpu.CompilerParams(
            dimension_semantics=("arbitrary",),
            vmem_limit_bytes=63 * 1024 * 1024,
        ),
    )(*([lap_matrix] * _SPLIT), features, W1, W2,
      b1.reshape(1, d), b2.reshape(1, d))
